# Initial kernel scaffold; baseline (speedup 1.0000x reference)
import jax
import jax.numpy as jnp
from jax.experimental import pallas as pl


def kernel(x, edge_index, edge_attr, W1, b1, W2, b2, Wlin, blin, Wa, ba, ln_g, ln_b, Wb, bb):
    raise NotImplementedError("write your pallas kernel here")


# speedup vs baseline: 2.6114x; 2.6114x over previous
import functools

import jax
import jax.numpy as jnp
from jax import lax
from jax.experimental import pallas as pl
from jax.experimental.pallas import tpu as pltpu
from jax.experimental.pallas import tpu_sc as plsc

N_NODES = 10000
N_EDGES = 320000
D = 128
HID = 128
EAD = 16

BE = 2000


def _edge_mlp_body(attr_ref, w1_ref, b1_ref, w2_ref, b2_ref, wl_ref, bl_ref,
                   out_ref):
  attr = attr_ref[...]
  pol = jnp.clip(attr[:, 0:1], 0.0, 1.0) + 0.01
  h = jnp.dot(attr, w1_ref[...], preferred_element_type=jnp.float32)
  h = jnp.maximum(h + b1_ref[...], 0.0)
  e = jnp.dot(h, w2_ref[...], preferred_element_type=jnp.float32)
  e = (e + b2_ref[...]) * pol
  o = jnp.dot(e, wl_ref[...], preferred_element_type=jnp.float32)
  out_ref[...] = o + bl_ref[...]


def _edge_mlp(edge_attr, w1p, b1, w2, b2, wlin, blin):
  n_blk = N_EDGES // BE
  wspec = lambda shape: pl.BlockSpec(shape, lambda i: (0, 0))
  return pl.pallas_call(
      _edge_mlp_body,
      grid=(n_blk,),
      in_specs=[
          pl.BlockSpec((BE, EAD), lambda i: (i, 0)),
          wspec((EAD, HID)),
          wspec((1, HID)),
          wspec((HID, HID)),
          wspec((1, HID)),
          wspec((HID, D)),
          wspec((1, D)),
      ],
      out_specs=pl.BlockSpec((BE, D), lambda i: (i, 0)),
      out_shape=jax.ShapeDtypeStruct((N_EDGES, D), jnp.float32),
  )(edge_attr, w1p, b1, w2, b2, wlin, blin)


_INFO = plsc.get_sparse_core_info()
NC = _INFO.num_cores
NS = _INFO.num_subcores
NW = NC * NS
EPW = N_EDGES // NW
KB = 80
NB = EPW // KB
NPAD = 10240
RPS = NPAD // NS


def _sc_body(x_hbm, src_hbm, dst_hbm, ea_hbm, zeros_hbm, out_hbm,
             idx_s, idx_d, xrows, earows, sem, agg_sh):
  c = lax.axis_index("c")
  s = lax.axis_index("s")
  wid = c * NS + s

  pltpu.sync_copy(zeros_hbm.at[pl.ds(s * RPS, RPS)],
                  agg_sh.at[pl.ds(s * RPS, RPS)])
  plsc.subcore_barrier()

  def block(b, carry):
    base = wid * EPW + b * KB
    pltpu.sync_copy(src_hbm.at[pl.ds(base, KB)], idx_s)
    pltpu.sync_copy(dst_hbm.at[pl.ds(base, KB)], idx_d)
    gather = pltpu.async_copy(x_hbm.at[idx_s], xrows, sem)
    pltpu.sync_copy(ea_hbm.at[pl.ds(base, KB)], earows)
    gather.wait()

    def row(i, carry2):
      for j in range(D // 16):
        a = xrows[i, pl.ds(j * 16, 16)]
        v = earows[i, pl.ds(j * 16, 16)]
        earows[i, pl.ds(j * 16, 16)] = jnp.maximum(a + v, 0.0)
      return carry2

    lax.fori_loop(0, KB, row, 0)
    pltpu.sync_copy(earows, agg_sh.at[idx_d], add=True)
    return carry

  lax.fori_loop(0, NB, block, 0)

  plsc.subcore_barrier()
  pltpu.sync_copy(agg_sh.at[pl.ds(s * RPS, RPS)],
                  out_hbm.at[c].at[pl.ds(s * RPS, RPS)])


def _sc_aggregate(x, src, dst, ea, zeros):
  mesh = plsc.VectorSubcoreMesh(core_axis_name="c", subcore_axis_name="s")
  f = pl.kernel(
      _sc_body,
      out_type=jax.ShapeDtypeStruct((NC, NPAD, D), jnp.float32),
      mesh=mesh,
      scratch_types=[
          pltpu.VMEM((KB,), jnp.int32),
          pltpu.VMEM((KB,), jnp.int32),
          pltpu.VMEM((KB, D), jnp.float32),
          pltpu.VMEM((KB, D), jnp.float32),
          pltpu.SemaphoreType.DMA,
          pltpu.VMEM_SHARED((NPAD, D), jnp.float32),
      ],
  )
  return f(x, src, dst, ea, zeros)


BN = 2000


def _node_mlp_body(p_ref, x_ref, wa_ref, ba_ref, g_ref, bt_ref, wb_ref,
                   bb_ref, out_ref):
  out = p_ref[0] + p_ref[1] + x_ref[...]
  h2 = jnp.dot(out, wa_ref[...], preferred_element_type=jnp.float32)
  h2 = h2 + ba_ref[...]
  mu = jnp.mean(h2, axis=-1, keepdims=True)
  d = h2 - mu
  var = jnp.mean(d * d, axis=-1, keepdims=True)
  h2 = d * lax.rsqrt(var + 1e-5) * g_ref[...] + bt_ref[...]
  h2 = jnp.maximum(h2, 0.0)
  o = jnp.dot(h2, wb_ref[...], preferred_element_type=jnp.float32)
  out_ref[...] = o + bb_ref[...]


def _node_mlp(partials, x, wa, ba, ln_g, ln_b, wb, bb):
  n_blk = N_NODES // BN
  wspec = lambda shape: pl.BlockSpec(shape, lambda i: (0, 0))
  return pl.pallas_call(
      _node_mlp_body,
      grid=(n_blk,),
      in_specs=[
          pl.BlockSpec((NC, BN, D), lambda i: (0, i, 0)),
          pl.BlockSpec((BN, D), lambda i: (i, 0)),
          wspec((D, D)),
          wspec((1, D)),
          wspec((1, D)),
          wspec((1, D)),
          wspec((D, D)),
          wspec((1, D)),
      ],
      out_specs=pl.BlockSpec((BN, D), lambda i: (i, 0)),
      out_shape=jax.ShapeDtypeStruct((N_NODES, D), jnp.float32),
  )(partials, x, wa, ba, ln_g, ln_b, wb, bb)


def kernel(x, edge_index, edge_attr, W1, b1, W2, b2, Wlin, blin, Wa, ba,
           ln_g, ln_b, Wb, bb):
  w1p = jnp.concatenate([jnp.zeros((1, HID), jnp.float32), W1], axis=0)
  ea = _edge_mlp(edge_attr, w1p, b1[None, :], W2, b2[None, :], Wlin,
                 blin[None, :])
  zeros = jnp.zeros((NPAD, D), jnp.float32)
  partials = _sc_aggregate(x, edge_index[0], edge_index[1], ea, zeros)
  return _node_mlp(partials, x, Wa, ba[None, :], ln_g[None, :],
                   ln_b[None, :], Wb, bb[None, :])


# speedup vs baseline: 3.6102x; 1.3825x over previous
import functools

import jax
import jax.numpy as jnp
from jax import lax
from jax.experimental import pallas as pl
from jax.experimental.pallas import tpu as pltpu
from jax.experimental.pallas import tpu_sc as plsc

N_NODES = 10000
N_EDGES = 320000
D = 128
HID = 128
EAD = 16

BE = 2000


def _edge_mlp_body(attr_ref, w1_ref, b1_ref, w2_ref, b2_ref, wl_ref, bl_ref,
                   out_ref):
  attr = attr_ref[...]
  pol = jnp.clip(attr[:, 0:1], 0.0, 1.0) + 0.01
  h = jnp.dot(attr, w1_ref[...], preferred_element_type=jnp.float32)
  h = jnp.maximum(h + b1_ref[...], 0.0)
  e = jnp.dot(h, w2_ref[...], preferred_element_type=jnp.float32)
  e = (e + b2_ref[...]) * pol
  o = jnp.dot(e, wl_ref[...], preferred_element_type=jnp.float32)
  out_ref[...] = o + bl_ref[...]


def _edge_mlp(edge_attr, w1p, b1, w2, b2, wlin, blin):
  n_blk = N_EDGES // BE
  wspec = lambda shape: pl.BlockSpec(shape, lambda i: (0, 0))
  return pl.pallas_call(
      _edge_mlp_body,
      grid=(n_blk,),
      in_specs=[
          pl.BlockSpec((BE, EAD), lambda i: (i, 0)),
          wspec((EAD, HID)),
          wspec((1, HID)),
          wspec((HID, HID)),
          wspec((1, HID)),
          wspec((HID, D)),
          wspec((1, D)),
      ],
      out_specs=pl.BlockSpec((BE, D), lambda i: (i, 0)),
      out_shape=jax.ShapeDtypeStruct((N_EDGES, D), jnp.float32),
  )(edge_attr, w1p, b1, w2, b2, wlin, blin)


_INFO = plsc.get_sparse_core_info()
NC = _INFO.num_cores
NS = _INFO.num_subcores
NW = NC * NS
EPW = N_EDGES // NW
KB = 80
NB = EPW // KB
NPAD = 10240
RPS = NPAD // NS


def _sc_body(x_hbm, src_hbm, dst_hbm, ea_hbm, zeros_hbm, out_hbm,
             si, di, xb, eb, ssi, sdi, sxb, seb, agg_sh):
  c = lax.axis_index("c")
  s = lax.axis_index("s")
  wid = c * NS + s
  ebase = wid * EPW

  pltpu.sync_copy(zeros_hbm.at[pl.ds(s * RPS, RPS)],
                  agg_sh.at[pl.ds(s * RPS, RPS)])
  plsc.subcore_barrier()

  def start_idx(b, k):
    base = ebase + b * KB
    pltpu.async_copy(src_hbm.at[pl.ds(base, KB)], si[k], ssi[k])
    pltpu.async_copy(dst_hbm.at[pl.ds(base, KB)], di[k], sdi[k])

  def wait_idx(k):
    pltpu.make_async_copy(src_hbm.at[pl.ds(0, KB)], si[k], ssi[k]).wait()
    pltpu.make_async_copy(dst_hbm.at[pl.ds(0, KB)], di[k], sdi[k]).wait()

  def start_data(b, k):
    pltpu.async_copy(x_hbm.at[si[k]], xb[k], sxb[k])
    pltpu.async_copy(ea_hbm.at[pl.ds(ebase + b * KB, KB)], eb[k], seb[k])

  def wait_data(k):
    pltpu.make_async_copy(x_hbm.at[pl.ds(0, KB)], xb[k], sxb[k]).wait()
    pltpu.make_async_copy(ea_hbm.at[pl.ds(0, KB)], eb[k], seb[k]).wait()

  def compute_scatter(k):
    xbuf, ebuf = xb[k], eb[k]

    @plsc.parallel_loop(0, KB, unroll=2)
    def _(i):
      for j in range(D // 16):
        a = xbuf[i, pl.ds(j * 16, 16)]
        v = ebuf[i, pl.ds(j * 16, 16)]
        ebuf[i, pl.ds(j * 16, 16)] = jnp.maximum(a + v, 0.0)

    pltpu.sync_copy(ebuf, agg_sh.at[di[k]], add=True)

  start_idx(0, 0)
  wait_idx(0)
  start_data(0, 0)
  start_idx(1, 1)

  def step(b, k):
    @pl.when(b + 1 < NB)
    def _():
      wait_idx(1 - k)
      start_data(b + 1, 1 - k)

    wait_data(k)
    compute_scatter(k)

    @pl.when(b + 2 < NB)
    def _():
      start_idx(b + 2, k)

  def pair(i, carry):
    step(2 * i, 0)

    @pl.when(2 * i + 1 < NB)
    def _():
      step(2 * i + 1, 1)

    return carry

  lax.fori_loop(0, (NB + 1) // 2, pair, 0)

  plsc.subcore_barrier()
  pltpu.sync_copy(agg_sh.at[pl.ds(s * RPS, RPS)],
                  out_hbm.at[c].at[pl.ds(s * RPS, RPS)])


def _sc_aggregate(x, src, dst, ea, zeros):
  mesh = plsc.VectorSubcoreMesh(core_axis_name="c", subcore_axis_name="s")
  f = pl.kernel(
      _sc_body,
      out_type=jax.ShapeDtypeStruct((NC, NPAD, D), jnp.float32),
      mesh=mesh,
      scratch_types=[
          [pltpu.VMEM((KB,), jnp.int32)] * 2,
          [pltpu.VMEM((KB,), jnp.int32)] * 2,
          [pltpu.VMEM((KB, D), jnp.float32)] * 2,
          [pltpu.VMEM((KB, D), jnp.float32)] * 2,
          [pltpu.SemaphoreType.DMA] * 2,
          [pltpu.SemaphoreType.DMA] * 2,
          [pltpu.SemaphoreType.DMA] * 2,
          [pltpu.SemaphoreType.DMA] * 2,
          pltpu.VMEM_SHARED((NPAD, D), jnp.float32),
      ],
  )
  return f(x, src, dst, ea, zeros)


BN = 2000


def _node_mlp_body(p_ref, x_ref, wa_ref, ba_ref, g_ref, bt_ref, wb_ref,
                   bb_ref, out_ref):
  out = p_ref[0] + p_ref[1] + x_ref[...]
  h2 = jnp.dot(out, wa_ref[...], preferred_element_type=jnp.float32)
  h2 = h2 + ba_ref[...]
  mu = jnp.mean(h2, axis=-1, keepdims=True)
  d = h2 - mu
  var = jnp.mean(d * d, axis=-1, keepdims=True)
  h2 = d * lax.rsqrt(var + 1e-5) * g_ref[...] + bt_ref[...]
  h2 = jnp.maximum(h2, 0.0)
  o = jnp.dot(h2, wb_ref[...], preferred_element_type=jnp.float32)
  out_ref[...] = o + bb_ref[...]


def _node_mlp(partials, x, wa, ba, ln_g, ln_b, wb, bb):
  n_blk = N_NODES // BN
  wspec = lambda shape: pl.BlockSpec(shape, lambda i: (0, 0))
  return pl.pallas_call(
      _node_mlp_body,
      grid=(n_blk,),
      in_specs=[
          pl.BlockSpec((NC, BN, D), lambda i: (0, i, 0)),
          pl.BlockSpec((BN, D), lambda i: (i, 0)),
          wspec((D, D)),
          wspec((1, D)),
          wspec((1, D)),
          wspec((1, D)),
          wspec((D, D)),
          wspec((1, D)),
      ],
      out_specs=pl.BlockSpec((BN, D), lambda i: (i, 0)),
      out_shape=jax.ShapeDtypeStruct((N_NODES, D), jnp.float32),
  )(partials, x, wa, ba, ln_g, ln_b, wb, bb)


def kernel(x, edge_index, edge_attr, W1, b1, W2, b2, Wlin, blin, Wa, ba,
           ln_g, ln_b, Wb, bb):
  w1p = jnp.concatenate([jnp.zeros((1, HID), jnp.float32), W1], axis=0)
  ea = _edge_mlp(edge_attr, w1p, b1[None, :], W2, b2[None, :], Wlin,
                 blin[None, :])
  zeros = jnp.zeros((NPAD, D), jnp.float32)
  partials = _sc_aggregate(x, edge_index[0], edge_index[1], ea, zeros)
  return _node_mlp(partials, x, Wa, ba[None, :], ln_g[None, :],
                   ln_b[None, :], Wb, bb[None, :])
